# Initial kernel scaffold; baseline (speedup 1.0000x reference)
import jax
import jax.numpy as jnp
from jax.experimental import pallas as pl


def kernel(x, codes, codebooks, scales, bias):
    raise NotImplementedError("write your pallas kernel here")


# speedup vs baseline: 26.7488x; 26.7488x over previous
import jax
import jax.numpy as jnp
from jax import lax
from jax.experimental import pallas as pl
from jax.experimental.pallas import tpu as pltpu
from jax.experimental.pallas import tpu_sc as plsc

OUT_F = 2048
IN_F = 2048
GROUPS = 256
GSIZE = 8
NW = 32
ROWS_PER_W = OUT_F // NW
VALS_PER_W = ROWS_PER_W * IN_F
CODES_PER_W = ROWS_PER_W * GROUPS
CHUNK_VALS = 16384
NCHUNK = VALS_PER_W // CHUNK_VALS
ITERS = CHUNK_VALS // 16


def _sc_dequant(codes0, codes1, t0, t1):
    mesh = plsc.VectorSubcoreMesh(core_axis_name="c", subcore_axis_name="s")

    def body(c0_hbm, c1_hbm, t0_hbm, t1_hbm, w_hbm, c0_v, c1_v, t0_v, t1_v, out_v):
        wid = lax.axis_index("s") * 2 + lax.axis_index("c")
        cbase = wid * CODES_PER_W
        vbase = wid * VALS_PER_W
        pltpu.sync_copy(t0_hbm, t0_v)
        pltpu.sync_copy(t1_hbm, t1_v)
        pltpu.sync_copy(c0_hbm.at[pl.ds(cbase, CODES_PER_W)], c0_v)
        pltpu.sync_copy(c1_hbm.at[pl.ds(cbase, CODES_PER_W)], c1_v)
        lane = lax.iota(jnp.int32, 16)
        hi = lane >> 3
        lane8 = lane & 7

        def chunk(c, carry):
            def it(i, carry2):
                v0 = i * 16
                vg = c * CHUNK_VALS + v0
                ridx = vg >> 11
                g0 = (vg & 2047) >> 3
                cidx = (ridx * GROUPS + g0) + hi
                cc0 = plsc.load_gather(c0_v, [cidx])
                cc1 = plsc.load_gather(c1_v, [cidx])
                a = plsc.load_gather(t0_v, [(cc0 << 3) + lane8])
                b = plsc.load_gather(t1_v, [(cc1 << 3) + lane8])
                out_v[pl.ds(v0, 16)] = a + b
                return carry2

            lax.fori_loop(0, ITERS, it, 0, unroll=4)
            pltpu.sync_copy(out_v, w_hbm.at[pl.ds(vbase + c * CHUNK_VALS, CHUNK_VALS)])
            return carry

        lax.fori_loop(0, NCHUNK, chunk, 0)

    f = pl.kernel(
        body,
        out_type=jax.ShapeDtypeStruct((OUT_F * IN_F,), jnp.float32),
        mesh=mesh,
        compiler_params=pltpu.CompilerParams(needs_layout_passes=False),
        scratch_types=[
            pltpu.VMEM((CODES_PER_W,), jnp.int32),
            pltpu.VMEM((CODES_PER_W,), jnp.int32),
            pltpu.VMEM((GROUPS * GSIZE,), jnp.float32),
            pltpu.VMEM((GROUPS * GSIZE,), jnp.float32),
            pltpu.VMEM((CHUNK_VALS,), jnp.float32),
        ],
    )
    return f(codes0, codes1, t0, t1)


def _mm_body(x_ref, w_ref, s_ref, b_ref, o_ref):
    acc = lax.dot_general(
        x_ref[...], w_ref[...].astype(jnp.bfloat16),
        (((1,), (1,)), ((), ())), preferred_element_type=jnp.float32)
    o_ref[...] = acc * s_ref[...] + b_ref[...]


def _tc_matmul(xb, w, s2, b2):
    m, k = xb.shape
    n = w.shape[0]
    bm, bn = 1024, 1024
    return pl.pallas_call(
        _mm_body,
        grid=(m // bm, n // bn),
        in_specs=[
            pl.BlockSpec((bm, k), lambda i, j: (i, 0)),
            pl.BlockSpec((bn, k), lambda i, j: (j, 0)),
            pl.BlockSpec((1, bn), lambda i, j: (0, j)),
            pl.BlockSpec((1, bn), lambda i, j: (0, j)),
        ],
        out_specs=pl.BlockSpec((bm, bn), lambda i, j: (i, j)),
        out_shape=jax.ShapeDtypeStruct((m, n), jnp.float32),
    )(xb, w, s2, b2)


def kernel(x, codes, codebooks, scales, bias):
    b, s, in_f = x.shape
    codes0 = codes[:, :, 0].reshape(-1)
    codes1 = codes[:, :, 1].reshape(-1)
    t0 = codebooks[0].reshape(-1)
    t1 = codebooks[1].reshape(-1)
    w_flat = _sc_dequant(codes0, codes1, t0, t1)
    w = w_flat.reshape(OUT_F, IN_F)
    xb = x.reshape(b * s, in_f).astype(jnp.bfloat16)
    s2 = scales.reshape(1, OUT_F)
    b2 = bias.reshape(1, OUT_F)
    out = _tc_matmul(xb, w, s2, b2)
    return out.reshape(b, s, OUT_F)
